# baseline (speedup 1.0000x reference)
import functools
import jax
import jax.numpy as jnp
from jax import lax
from jax.experimental import pallas as pl
from jax.experimental.pallas import tpu as pltpu
from jax.experimental.pallas import tpu_sc as plsc

N_NODES = 10000
N_EDGES = 5000
NNZ = 160000
IN_CH = 128
HIDDEN = 256
HALF = HIDDEN // 2

NC = 2
NS = 16
L = 16

F32 = jnp.float32
I32 = jnp.int32

_mesh = lambda: plsc.VectorSubcoreMesh(
    core_axis_name="c", subcore_axis_name="s", num_cores=NC, num_subcores=NS)

_SC_PARAMS = pltpu.CompilerParams(needs_layout_passes=False)


def _make_deg_kernel():
    P = NNZ // (NC * NS)
    FULL = P // L
    REM = P - FULL * L
    PP = (FULL + 1) * L if REM else P

    @functools.partial(
        pl.kernel,
        out_type=(
            jax.ShapeDtypeStruct((NC * NS, N_NODES), F32),
            jax.ShapeDtypeStruct((NC * NS, N_EDGES), F32),
        ),
        mesh=_mesh(),
        compiler_params=_SC_PARAMS,
        scratch_types=[
            pltpu.VMEM((PP,), I32),
            pltpu.VMEM((PP,), I32),
            pltpu.VMEM((N_NODES,), F32),
            pltpu.VMEM((N_EDGES,), F32),
        ],
    )
    def deg_kernel(ni_hbm, ei_hbm, nd_out, ed_out, ni_v, ei_v, nd_acc, ed_acc):
        w = lax.axis_index("c") * NS + lax.axis_index("s")
        base = w * P
        if REM:
            zi = jnp.zeros((L,), I32)
            ni_v[pl.ds(FULL * L, L)] = zi
            ei_v[pl.ds(FULL * L, L)] = zi
        pltpu.sync_copy(ni_hbm.at[pl.ds(base, P)], ni_v.at[pl.ds(0, P)])
        pltpu.sync_copy(ei_hbm.at[pl.ds(base, P)], ei_v.at[pl.ds(0, P)])

        zeros = jnp.zeros((L,), F32)

        @pl.loop(0, N_NODES // L)
        def _(i):
            nd_acc[pl.ds(i * L, L)] = zeros

        @pl.loop(0, N_EDGES // L)
        def _(i):
            ed_acc[pl.ds(i * L, L)] = zeros

        ones = jnp.ones((L,), F32)

        @pl.loop(0, FULL)
        def _(j):
            ni = ni_v[pl.ds(j * L, L)]
            ei = ei_v[pl.ds(j * L, L)]
            plsc.addupdate_scatter(nd_acc, [ni], ones)
            plsc.addupdate_scatter(ed_acc, [ei], ones)

        if REM:
            m = lax.iota(I32, L) < REM
            ni = ni_v[pl.ds(FULL * L, L)]
            ei = ei_v[pl.ds(FULL * L, L)]
            plsc.addupdate_scatter(nd_acc, [ni], ones, mask=m)
            plsc.addupdate_scatter(ed_acc, [ei], ones, mask=m)

        pltpu.sync_copy(nd_acc, nd_out.at[w])
        pltpu.sync_copy(ed_acc, ed_out.at[w])

    return deg_kernel


def _make_norm_kernel():
    P = NNZ // (NC * NS)
    FULL = P // L
    REM = P - FULL * L
    PP = (FULL + 1) * L if REM else P

    @functools.partial(
        pl.kernel,
        out_type=(
            jax.ShapeDtypeStruct((NC * NS, N_NODES), F32),
            jax.ShapeDtypeStruct((NC * NS, N_EDGES), F32),
        ),
        mesh=_mesh(),
        compiler_params=_SC_PARAMS,
        scratch_types=[
            pltpu.VMEM((PP,), I32),
            pltpu.VMEM((PP,), I32),
            pltpu.VMEM((N_NODES,), F32),
            pltpu.VMEM((N_EDGES,), F32),
            pltpu.VMEM((N_NODES,), F32),
            pltpu.VMEM((N_EDGES,), F32),
        ],
    )
    def norm_kernel(ni_hbm, ei_hbm, ncard_hbm, ecard_hbm, s0_out, s1_out,
                    ni_v, ei_v, nc_v, ec_v, s0_acc, s1_acc):
        w = lax.axis_index("c") * NS + lax.axis_index("s")
        base = w * P
        if REM:
            zi = jnp.zeros((L,), I32)
            ni_v[pl.ds(FULL * L, L)] = zi
            ei_v[pl.ds(FULL * L, L)] = zi
        pltpu.sync_copy(ni_hbm.at[pl.ds(base, P)], ni_v.at[pl.ds(0, P)])
        pltpu.sync_copy(ei_hbm.at[pl.ds(base, P)], ei_v.at[pl.ds(0, P)])
        pltpu.sync_copy(ncard_hbm, nc_v)
        pltpu.sync_copy(ecard_hbm, ec_v)

        zeros = jnp.zeros((L,), F32)

        @pl.loop(0, N_NODES // L)
        def _(i):
            s0_acc[pl.ds(i * L, L)] = zeros

        @pl.loop(0, N_EDGES // L)
        def _(i):
            s1_acc[pl.ds(i * L, L)] = zeros

        @pl.loop(0, FULL)
        def _(j):
            ni = ni_v[pl.ds(j * L, L)]
            ei = ei_v[pl.ds(j * L, L)]
            ec = plsc.load_gather(ec_v, [ei])
            plsc.addupdate_scatter(s0_acc, [ni], ec)
            nc = plsc.load_gather(nc_v, [ni])
            plsc.addupdate_scatter(s1_acc, [ei], nc)

        if REM:
            m = lax.iota(I32, L) < REM
            ni = ni_v[pl.ds(FULL * L, L)]
            ei = ei_v[pl.ds(FULL * L, L)]
            ec = plsc.load_gather(ec_v, [ei], mask=m)
            plsc.addupdate_scatter(s0_acc, [ni], ec, mask=m)
            nc = plsc.load_gather(nc_v, [ni], mask=m)
            plsc.addupdate_scatter(s1_acc, [ei], nc, mask=m)

        pltpu.sync_copy(s0_acc, s0_out.at[w])
        pltpu.sync_copy(s1_acc, s1_out.at[w])

    return norm_kernel


def _make_pass_kernel(n_src, n_dst_pad, chunk):
    P = NNZ // NS
    assert P % chunk == 0
    NCHUNK = P // chunk
    NBUF = 3
    GROUPS = NCHUNK // NBUF
    TAIL = NCHUNK - GROUPS * NBUF
    R = n_dst_pad // NS
    G = chunk
    assert R % G == 0

    @functools.partial(
        pl.kernel,
        out_type=jax.ShapeDtypeStruct((NC, n_dst_pad, HALF), F32),
        mesh=_mesh(),
        compiler_params=_SC_PARAMS,
        scratch_types=[
            pltpu.VMEM((P,), I32),
            pltpu.VMEM((NBUF, chunk), I32),
            pltpu.VMEM((NBUF, chunk, HALF), F32),
            pltpu.VMEM_SHARED((n_dst_pad, HALF), F32),
        ] + [pltpu.SemaphoreType.DMA] * (3 * NBUF),
    )
    def pass_kernel(table_hbm, src_hbm, dst_hbm, out_hbm,
                    si_v, di_v, rows_v, acc, *sems):
        sem_g = sems[:NBUF]
        sem_i = sems[NBUF:2 * NBUF]
        sem_s = sems[2 * NBUF:]
        c = lax.axis_index("c")
        s = lax.axis_index("s")
        row_off = (c * n_src).astype(I32)

        pltpu.sync_copy(src_hbm.at[pl.ds(s * P, P)], si_v)

        @pl.loop(0, P // L)
        def _(i):
            si_v[pl.ds(i * L, L)] = si_v[pl.ds(i * L, L)] + row_off

        zeros = jnp.zeros((L,), F32)

        @pl.loop(0, G)
        def _(i):
            for k in range(HALF // L):
                rows_v[0, i, pl.ds(k * L, L)] = zeros

        @pl.loop(0, R // G)
        def _(i):
            pltpu.sync_copy(rows_v.at[0].at[pl.ds(0, G)],
                            acc.at[pl.ds(s * R + i * G, G)])

        plsc.subcore_barrier()

        def fire(j, b):
            pltpu.async_copy(dst_hbm.at[s].at[j], di_v.at[b], sem_i[b])
            pltpu.async_copy(
                table_hbm.at[si_v.at[pl.ds(j * chunk, chunk)]],
                rows_v.at[b], sem_g[b])

        def wait_in(b):
            pltpu.make_async_copy(dst_hbm.at[s].at[0], di_v.at[b],
                                  sem_i[b]).wait()
            pltpu.make_async_copy(
                table_hbm.at[si_v.at[pl.ds(0, chunk)]],
                rows_v.at[b], sem_g[b]).wait()

        def fire_scat(b):
            pltpu.async_copy(rows_v.at[b], acc.at[di_v.at[b]], sem_s[b],
                             add=True)

        def wait_scat(b):
            pltpu.make_async_copy(rows_v.at[b], acc.at[di_v.at[b]],
                                  sem_s[b]).wait()

        for b in range(NBUF):
            fire(b, b)

        @pl.loop(0, GROUPS)
        def _(g):
            for b in range(NBUF):
                wait_in(b)
                fire_scat(b)
            for b in range(NBUF):
                j = g * NBUF + b
                wait_scat(b)

                @pl.when(j + NBUF < NCHUNK)
                def _():
                    fire(j + NBUF, b)

        for t in range(TAIL):
            wait_in(t)
            fire_scat(t)
        for t in range(TAIL):
            wait_scat(t)

        plsc.subcore_barrier()

        @pl.loop(0, R // G)
        def _(i):
            pltpu.sync_copy(acc.at[pl.ds(s * R + i * G, G)],
                            out_hbm.at[c, pl.ds(s * R + i * G, G)])

    return pass_kernel


_DOT = dict(precision=lax.Precision.HIGHEST, preferred_element_type=F32)


def _cards_call(nd_p, ed_p):
    def body(nd_ref, ed_ref, ncard_ref, ecard_ref):
        nd = jnp.sum(nd_ref[...], axis=0, keepdims=True)
        ed = jnp.sum(ed_ref[...], axis=0, keepdims=True)
        nd = jnp.where(nd > 0, nd, 1.0)
        ed = jnp.where(ed > 0, ed, 1.0)
        ncard_ref[...] = lax.rsqrt(nd)
        r = lax.rsqrt(ed)
        ecard_ref[...] = r * r * r

    return pl.pallas_call(
        body,
        out_shape=(jax.ShapeDtypeStruct((1, N_NODES), F32),
                   jax.ShapeDtypeStruct((1, N_EDGES), F32)),
    )(nd_p, ed_p)


def _dinv_call(s0_p, s1_p):
    def body(s0_ref, s1_ref, d0_ref, d1_ref):
        s0 = jnp.sum(s0_ref[...], axis=0, keepdims=True)
        s1 = jnp.sum(s1_ref[...], axis=0, keepdims=True)
        d0_ref[...] = jnp.where(s0 > 0, 1.0 / jnp.where(s0 > 0, s0, 1.0), 0.0)
        d1_ref[...] = jnp.where(s1 > 0, 1.0 / jnp.where(s1 > 0, s1, 1.0), 0.0)

    return pl.pallas_call(
        body,
        out_shape=(jax.ShapeDtypeStruct((1, N_NODES), F32),
                   jax.ShapeDtypeStruct((1, N_EDGES), F32)),
    )(s0_p, s1_p)


def _prep_call(x0, w0, ncard):
    n, k = x0.shape
    blk = 2000
    grid = n // blk

    def body(x_ref, w_ref, c_ref, o_ref):
        m = lax.dot_general(x_ref[...], w_ref[...],
                            (((1,), (0,)), ((), ())), **_DOT)
        m = m * c_ref[...]
        o_ref[0] = m[:, :HALF]
        o_ref[1] = m[:, HALF:]

    return pl.pallas_call(
        body,
        grid=(grid,),
        in_specs=[
            pl.BlockSpec((blk, k), lambda i: (i, 0)),
            pl.BlockSpec((k, HIDDEN), lambda i: (0, 0)),
            pl.BlockSpec((blk, 1), lambda i: (i, 0)),
        ],
        out_specs=pl.BlockSpec((2, blk, HALF), lambda i: (0, i, 0)),
        out_shape=jax.ShapeDtypeStruct((2, n, HALF), F32),
    )(x0, w0, ncard)


def _mid_call(agg, dinv, b, w, card, blk):
    n = dinv.shape[0]
    grid = n // blk

    def body(a0_ref, a1_ref, di_ref, b_ref, w_ref, c_ref, o_ref):
        a = jnp.concatenate([a0_ref[0], a1_ref[0]], axis=1)
        x = jax.nn.sigmoid(a * di_ref[...] + b_ref[...])
        m = lax.dot_general(x, w_ref[...], (((1,), (0,)), ((), ())), **_DOT)
        m = m * c_ref[...]
        o_ref[0] = m[:, :HALF]
        o_ref[1] = m[:, HALF:]

    return pl.pallas_call(
        body,
        grid=(grid,),
        in_specs=[
            pl.BlockSpec((1, blk, HALF), lambda i: (0, i, 0)),
            pl.BlockSpec((1, blk, HALF), lambda i: (1, i, 0)),
            pl.BlockSpec((blk, 1), lambda i: (i, 0)),
            pl.BlockSpec((1, HIDDEN), lambda i: (0, 0)),
            pl.BlockSpec((HIDDEN, HIDDEN), lambda i: (0, 0)),
            pl.BlockSpec((blk, 1), lambda i: (i, 0)),
        ],
        out_specs=pl.BlockSpec((2, blk, HALF), lambda i: (0, i, 0)),
        out_shape=jax.ShapeDtypeStruct((2, n, HALF), F32),
    )(agg, agg, dinv, b, w, card)


def _final_call(agg, dinv, b, w_lin, b_lin):
    n = dinv.shape[0]
    blk = 2000
    grid = n // blk

    def body(a0_ref, a1_ref, di_ref, b_ref, wl_ref, bl_ref, o_ref, acc_ref):
        a = jnp.concatenate([a0_ref[0], a1_ref[0]], axis=1)
        x = jax.nn.sigmoid(a * di_ref[...] + b_ref[...])
        bmax = jnp.max(x, axis=0, keepdims=True)
        i = pl.program_id(0)

        @pl.when(i == 0)
        def _():
            acc_ref[...] = bmax

        @pl.when(i > 0)
        def _():
            acc_ref[...] = jnp.maximum(acc_ref[...], bmax)

        @pl.when(i == grid - 1)
        def _():
            o_ref[...] = lax.dot_general(
                acc_ref[...], wl_ref[...], (((1,), (0,)), ((), ())),
                **_DOT) + bl_ref[...]

    return pl.pallas_call(
        body,
        grid=(grid,),
        in_specs=[
            pl.BlockSpec((1, blk, HALF), lambda i: (0, i, 0)),
            pl.BlockSpec((1, blk, HALF), lambda i: (1, i, 0)),
            pl.BlockSpec((blk, 1), lambda i: (i, 0)),
            pl.BlockSpec((1, HIDDEN), lambda i: (0, 0)),
            pl.BlockSpec((HIDDEN, 1), lambda i: (0, 0)),
            pl.BlockSpec((1, 1), lambda i: (0, 0)),
        ],
        out_specs=pl.BlockSpec((1, 1), lambda i: (0, 0)),
        out_shape=jax.ShapeDtypeStruct((1, 1), F32),
        scratch_shapes=[pltpu.VMEM((1, HIDDEN), F32)],
    )(agg, agg, dinv, b, w_lin, b_lin)


N_NODES_PAD = 10240
N_EDGES_PAD = 5120
CHUNK = 80

_deg = _make_deg_kernel()
_norm_sc = _make_norm_kernel()
_pass_n2e = _make_pass_kernel(N_NODES, N_EDGES_PAD, CHUNK)
_pass_e2n = _make_pass_kernel(N_EDGES, N_NODES_PAD, CHUNK)


@jax.jit
def kernel(x_0, node_idx, edge_idx, W0_1, W1_1, b01_1, b10_1,
           W0_2, W1_2, b01_2, b10_2, W_lin, b_lin):
    nd_p, ed_p = _deg(node_idx, edge_idx)
    ncard, ecard = _cards_call(nd_p, ed_p)
    s0_p, s1_p = _norm_sc(node_idx, edge_idx,
                          ncard.reshape(N_NODES), ecard.reshape(N_EDGES))
    d0_inv, d1_inv = _dinv_call(s0_p, s1_p)
    ncard = ncard.reshape(N_NODES, 1)
    ecard = ecard.reshape(N_EDGES, 1)
    d0_inv = d0_inv.reshape(N_NODES, 1)
    d1_inv = d1_inv.reshape(N_EDGES, 1)

    ei2 = edge_idx.reshape(NS, NNZ // NS // CHUNK, CHUNK)
    ni2 = node_idx.reshape(NS, NNZ // NS // CHUNK, CHUNK)

    b01_1r = b01_1.reshape(1, HIDDEN)
    b10_1r = b10_1.reshape(1, HIDDEN)
    b01_2r = b01_2.reshape(1, HIDDEN)
    b10_2r = b10_2.reshape(1, HIDDEN)

    t1 = _prep_call(x_0, W0_1, ncard).reshape(2 * N_NODES, HALF)
    agg_e = _pass_n2e(t1, node_idx, ei2)
    t2 = _mid_call(agg_e, d1_inv, b01_1r, W1_1, ecard, 1000)
    t2 = t2.reshape(2 * N_EDGES, HALF)
    agg_v = _pass_e2n(t2, edge_idx, ni2)

    t3 = _mid_call(agg_v, d0_inv, b10_1r, W0_2, ncard, 2000)
    t3 = t3.reshape(2 * N_NODES, HALF)
    agg_e2 = _pass_n2e(t3, node_idx, ei2)
    t4 = _mid_call(agg_e2, d1_inv, b01_2r, W1_2, ecard, 1000)
    t4 = t4.reshape(2 * N_EDGES, HALF)
    agg_v2 = _pass_e2n(t4, edge_idx, ni2)

    out = _final_call(agg_v2, d0_inv, b10_2r, W_lin, b_lin.reshape(1, 1))
    return out.reshape(1)


# speedup vs baseline: 1.1123x; 1.1123x over previous
import functools
import jax
import jax.numpy as jnp
from jax import lax
from jax.experimental import pallas as pl
from jax.experimental.pallas import tpu as pltpu
from jax.experimental.pallas import tpu_sc as plsc

N_NODES = 10000
N_EDGES = 5000
NNZ = 160000
IN_CH = 128
HIDDEN = 256
HALF = HIDDEN // 2

NC = 2
NS = 16
L = 16

F32 = jnp.float32
I32 = jnp.int32

_mesh = lambda: plsc.VectorSubcoreMesh(
    core_axis_name="c", subcore_axis_name="s", num_cores=NC, num_subcores=NS)

_SC_PARAMS = pltpu.CompilerParams(needs_layout_passes=False)


def _make_deg_kernel():
    P = NNZ // (NC * NS)
    FULL = P // L
    REM = P - FULL * L
    PP = (FULL + 1) * L if REM else P

    @functools.partial(
        pl.kernel,
        out_type=(
            jax.ShapeDtypeStruct((NC * NS, N_NODES), F32),
            jax.ShapeDtypeStruct((NC * NS, N_EDGES), F32),
        ),
        mesh=_mesh(),
        compiler_params=_SC_PARAMS,
        scratch_types=[
            pltpu.VMEM((PP,), I32),
            pltpu.VMEM((PP,), I32),
            pltpu.VMEM((N_NODES,), F32),
            pltpu.VMEM((N_EDGES,), F32),
        ],
    )
    def deg_kernel(ni_hbm, ei_hbm, nd_out, ed_out, ni_v, ei_v, nd_acc, ed_acc):
        w = lax.axis_index("c") * NS + lax.axis_index("s")
        base = w * P
        if REM:
            zi = jnp.zeros((L,), I32)
            ni_v[pl.ds(FULL * L, L)] = zi
            ei_v[pl.ds(FULL * L, L)] = zi
        pltpu.sync_copy(ni_hbm.at[pl.ds(base, P)], ni_v.at[pl.ds(0, P)])
        pltpu.sync_copy(ei_hbm.at[pl.ds(base, P)], ei_v.at[pl.ds(0, P)])

        zeros = jnp.zeros((L,), F32)

        @pl.loop(0, N_NODES // L)
        def _(i):
            nd_acc[pl.ds(i * L, L)] = zeros

        @pl.loop(0, N_EDGES // L)
        def _(i):
            ed_acc[pl.ds(i * L, L)] = zeros

        ones = jnp.ones((L,), F32)

        @pl.loop(0, FULL)
        def _(j):
            ni = ni_v[pl.ds(j * L, L)]
            ei = ei_v[pl.ds(j * L, L)]
            plsc.addupdate_scatter(nd_acc, [ni], ones)
            plsc.addupdate_scatter(ed_acc, [ei], ones)

        if REM:
            m = lax.iota(I32, L) < REM
            ni = ni_v[pl.ds(FULL * L, L)]
            ei = ei_v[pl.ds(FULL * L, L)]
            plsc.addupdate_scatter(nd_acc, [ni], ones, mask=m)
            plsc.addupdate_scatter(ed_acc, [ei], ones, mask=m)

        pltpu.sync_copy(nd_acc, nd_out.at[w])
        pltpu.sync_copy(ed_acc, ed_out.at[w])

    return deg_kernel


def _make_norm_kernel():
    P = NNZ // (NC * NS)
    FULL = P // L
    REM = P - FULL * L
    PP = (FULL + 1) * L if REM else P

    @functools.partial(
        pl.kernel,
        out_type=(
            jax.ShapeDtypeStruct((NC * NS, N_NODES), F32),
            jax.ShapeDtypeStruct((NC * NS, N_EDGES), F32),
        ),
        mesh=_mesh(),
        compiler_params=_SC_PARAMS,
        scratch_types=[
            pltpu.VMEM((PP,), I32),
            pltpu.VMEM((PP,), I32),
            pltpu.VMEM((N_NODES,), F32),
            pltpu.VMEM((N_EDGES,), F32),
            pltpu.VMEM((N_NODES,), F32),
            pltpu.VMEM((N_EDGES,), F32),
        ],
    )
    def norm_kernel(ni_hbm, ei_hbm, ncard_hbm, ecard_hbm, s0_out, s1_out,
                    ni_v, ei_v, nc_v, ec_v, s0_acc, s1_acc):
        w = lax.axis_index("c") * NS + lax.axis_index("s")
        base = w * P
        if REM:
            zi = jnp.zeros((L,), I32)
            ni_v[pl.ds(FULL * L, L)] = zi
            ei_v[pl.ds(FULL * L, L)] = zi
        pltpu.sync_copy(ni_hbm.at[pl.ds(base, P)], ni_v.at[pl.ds(0, P)])
        pltpu.sync_copy(ei_hbm.at[pl.ds(base, P)], ei_v.at[pl.ds(0, P)])
        pltpu.sync_copy(ncard_hbm, nc_v)
        pltpu.sync_copy(ecard_hbm, ec_v)

        zeros = jnp.zeros((L,), F32)

        @pl.loop(0, N_NODES // L)
        def _(i):
            s0_acc[pl.ds(i * L, L)] = zeros

        @pl.loop(0, N_EDGES // L)
        def _(i):
            s1_acc[pl.ds(i * L, L)] = zeros

        @pl.loop(0, FULL)
        def _(j):
            ni = ni_v[pl.ds(j * L, L)]
            ei = ei_v[pl.ds(j * L, L)]
            ec = plsc.load_gather(ec_v, [ei])
            plsc.addupdate_scatter(s0_acc, [ni], ec)
            nc = plsc.load_gather(nc_v, [ni])
            plsc.addupdate_scatter(s1_acc, [ei], nc)

        if REM:
            m = lax.iota(I32, L) < REM
            ni = ni_v[pl.ds(FULL * L, L)]
            ei = ei_v[pl.ds(FULL * L, L)]
            ec = plsc.load_gather(ec_v, [ei], mask=m)
            plsc.addupdate_scatter(s0_acc, [ni], ec, mask=m)
            nc = plsc.load_gather(nc_v, [ni], mask=m)
            plsc.addupdate_scatter(s1_acc, [ei], nc, mask=m)

        pltpu.sync_copy(s0_acc, s0_out.at[w])
        pltpu.sync_copy(s1_acc, s1_out.at[w])

    return norm_kernel


def _make_pass_kernel(n_src, n_dst_pad, chunk):
    P = NNZ // NS
    NBUF = 2
    NCHUNK = P // chunk
    TAIL = P - NCHUNK * chunk
    GROUPS = NCHUNK // NBUF
    assert GROUPS * NBUF == NCHUNK and TAIL % 8 == 0 and TAIL < chunk
    R = n_dst_pad // NS
    G = 80
    assert R % G == 0

    @functools.partial(
        pl.kernel,
        out_type=jax.ShapeDtypeStruct((NC, n_dst_pad, HALF), F32),
        mesh=_mesh(),
        compiler_params=_SC_PARAMS,
        scratch_types=[
            pltpu.VMEM((P,), I32),
            pltpu.VMEM((NBUF, chunk), I32),
            pltpu.VMEM((NBUF, chunk, HALF), F32),
            pltpu.VMEM_SHARED((n_dst_pad, HALF), F32),
        ] + [pltpu.SemaphoreType.DMA] * (2 * NBUF),
    )
    def pass_kernel(table_hbm, src_hbm, dst_hbm, out_hbm,
                    si_v, di_v, rows_v, acc, *sems):
        sem_g = sems[:NBUF]
        sem_i = sems[NBUF:]
        c = lax.axis_index("c")
        s = lax.axis_index("s")
        row_off = (c * n_src).astype(I32)

        pltpu.sync_copy(src_hbm.at[pl.ds(s * P, P)], si_v)

        @pl.loop(0, P // L)
        def _(i):
            si_v[pl.ds(i * L, L)] = si_v[pl.ds(i * L, L)] + row_off

        zeros = jnp.zeros((L,), F32)

        @pl.loop(0, G)
        def _(i):
            for k in range(HALF // L):
                rows_v[0, i, pl.ds(k * L, L)] = zeros

        @pl.loop(0, R // G)
        def _(i):
            pltpu.sync_copy(rows_v.at[0].at[pl.ds(0, G)],
                            acc.at[pl.ds(s * R + i * G, G)])

        plsc.subcore_barrier()

        def fire(j, b, n=chunk):
            pltpu.async_copy(dst_hbm.at[pl.ds(s * P + j * chunk, n)],
                             di_v.at[b].at[pl.ds(0, n)], sem_i[b])
            pltpu.async_copy(
                table_hbm.at[si_v.at[pl.ds(j * chunk, n)]],
                rows_v.at[b].at[pl.ds(0, n)], sem_g[b])

        def wait_in(b, n=chunk):
            pltpu.make_async_copy(dst_hbm.at[pl.ds(s * P, n)],
                                  di_v.at[b].at[pl.ds(0, n)], sem_i[b]).wait()
            pltpu.make_async_copy(
                table_hbm.at[si_v.at[pl.ds(0, n)]],
                rows_v.at[b].at[pl.ds(0, n)], sem_g[b]).wait()

        def scat(b, n=chunk):
            pltpu.sync_copy(rows_v.at[b].at[pl.ds(0, n)],
                            acc.at[di_v.at[b].at[pl.ds(0, n)]], add=True)

        for b in range(NBUF):
            fire(b, b)

        @pl.loop(0, GROUPS)
        def _(g):
            for b in range(NBUF):
                j = g * NBUF + b
                wait_in(b)
                scat(b)

                @pl.when(j + NBUF < NCHUNK)
                def _():
                    fire(j + NBUF, b)

        if TAIL:
            fire(NCHUNK, 0, TAIL)
            wait_in(0, TAIL)
            scat(0, TAIL)

        plsc.subcore_barrier()

        @pl.loop(0, R // G)
        def _(i):
            pltpu.sync_copy(acc.at[pl.ds(s * R + i * G, G)],
                            out_hbm.at[c, pl.ds(s * R + i * G, G)])

    return pass_kernel


_DOT = dict(precision=lax.Precision.HIGHEST, preferred_element_type=F32)


def _cards_call(nd_p, ed_p):
    def body(nd_ref, ed_ref, ncard_ref, ecard_ref):
        nd = jnp.sum(nd_ref[...], axis=0, keepdims=True)
        ed = jnp.sum(ed_ref[...], axis=0, keepdims=True)
        nd = jnp.where(nd > 0, nd, 1.0)
        ed = jnp.where(ed > 0, ed, 1.0)
        ncard_ref[...] = lax.rsqrt(nd)
        r = lax.rsqrt(ed)
        ecard_ref[...] = r * r * r

    return pl.pallas_call(
        body,
        out_shape=(jax.ShapeDtypeStruct((1, N_NODES), F32),
                   jax.ShapeDtypeStruct((1, N_EDGES), F32)),
    )(nd_p, ed_p)


def _dinv_call(s0_p, s1_p):
    def body(s0_ref, s1_ref, d0_ref, d1_ref):
        s0 = jnp.sum(s0_ref[...], axis=0, keepdims=True)
        s1 = jnp.sum(s1_ref[...], axis=0, keepdims=True)
        d0_ref[...] = jnp.where(s0 > 0, 1.0 / jnp.where(s0 > 0, s0, 1.0), 0.0)
        d1_ref[...] = jnp.where(s1 > 0, 1.0 / jnp.where(s1 > 0, s1, 1.0), 0.0)

    return pl.pallas_call(
        body,
        out_shape=(jax.ShapeDtypeStruct((1, N_NODES), F32),
                   jax.ShapeDtypeStruct((1, N_EDGES), F32)),
    )(s0_p, s1_p)


def _prep_call(x0, w0, ncard):
    n, k = x0.shape
    blk = 2000
    grid = n // blk

    def body(x_ref, w_ref, c_ref, o_ref):
        m = lax.dot_general(x_ref[...], w_ref[...],
                            (((1,), (0,)), ((), ())), **_DOT)
        m = m * c_ref[...]
        o_ref[0] = m[:, :HALF]
        o_ref[1] = m[:, HALF:]

    return pl.pallas_call(
        body,
        grid=(grid,),
        in_specs=[
            pl.BlockSpec((blk, k), lambda i: (i, 0)),
            pl.BlockSpec((k, HIDDEN), lambda i: (0, 0)),
            pl.BlockSpec((blk, 1), lambda i: (i, 0)),
        ],
        out_specs=pl.BlockSpec((2, blk, HALF), lambda i: (0, i, 0)),
        out_shape=jax.ShapeDtypeStruct((2, n, HALF), F32),
    )(x0, w0, ncard)


def _mid_call(agg, dinv, b, w, card, blk):
    n = dinv.shape[0]
    grid = n // blk

    def body(a0_ref, a1_ref, di_ref, b_ref, w_ref, c_ref, o_ref):
        a = jnp.concatenate([a0_ref[0], a1_ref[0]], axis=1)
        x = jax.nn.sigmoid(a * di_ref[...] + b_ref[...])
        m = lax.dot_general(x, w_ref[...], (((1,), (0,)), ((), ())), **_DOT)
        m = m * c_ref[...]
        o_ref[0] = m[:, :HALF]
        o_ref[1] = m[:, HALF:]

    return pl.pallas_call(
        body,
        grid=(grid,),
        in_specs=[
            pl.BlockSpec((1, blk, HALF), lambda i: (0, i, 0)),
            pl.BlockSpec((1, blk, HALF), lambda i: (1, i, 0)),
            pl.BlockSpec((blk, 1), lambda i: (i, 0)),
            pl.BlockSpec((1, HIDDEN), lambda i: (0, 0)),
            pl.BlockSpec((HIDDEN, HIDDEN), lambda i: (0, 0)),
            pl.BlockSpec((blk, 1), lambda i: (i, 0)),
        ],
        out_specs=pl.BlockSpec((2, blk, HALF), lambda i: (0, i, 0)),
        out_shape=jax.ShapeDtypeStruct((2, n, HALF), F32),
    )(agg, agg, dinv, b, w, card)


def _final_call(agg, dinv, b, w_lin, b_lin):
    n = dinv.shape[0]
    blk = 2000
    grid = n // blk

    def body(a0_ref, a1_ref, di_ref, b_ref, wl_ref, bl_ref, o_ref, acc_ref):
        a = jnp.concatenate([a0_ref[0], a1_ref[0]], axis=1)
        x = jax.nn.sigmoid(a * di_ref[...] + b_ref[...])
        bmax = jnp.max(x, axis=0, keepdims=True)
        i = pl.program_id(0)

        @pl.when(i == 0)
        def _():
            acc_ref[...] = bmax

        @pl.when(i > 0)
        def _():
            acc_ref[...] = jnp.maximum(acc_ref[...], bmax)

        @pl.when(i == grid - 1)
        def _():
            o_ref[...] = lax.dot_general(
                acc_ref[...], wl_ref[...], (((1,), (0,)), ((), ())),
                **_DOT) + bl_ref[...]

    return pl.pallas_call(
        body,
        grid=(grid,),
        in_specs=[
            pl.BlockSpec((1, blk, HALF), lambda i: (0, i, 0)),
            pl.BlockSpec((1, blk, HALF), lambda i: (1, i, 0)),
            pl.BlockSpec((blk, 1), lambda i: (i, 0)),
            pl.BlockSpec((1, HIDDEN), lambda i: (0, 0)),
            pl.BlockSpec((HIDDEN, 1), lambda i: (0, 0)),
            pl.BlockSpec((1, 1), lambda i: (0, 0)),
        ],
        out_specs=pl.BlockSpec((1, 1), lambda i: (0, 0)),
        out_shape=jax.ShapeDtypeStruct((1, 1), F32),
        scratch_shapes=[pltpu.VMEM((1, HIDDEN), F32)],
    )(agg, agg, dinv, b, w_lin, b_lin)


N_NODES_PAD = 10240
N_EDGES_PAD = 5120
CHUNK = 128

_deg = _make_deg_kernel()
_norm_sc = _make_norm_kernel()
_pass_n2e = _make_pass_kernel(N_NODES, N_EDGES_PAD, CHUNK)
_pass_e2n = _make_pass_kernel(N_EDGES, N_NODES_PAD, CHUNK)


@jax.jit
def kernel(x_0, node_idx, edge_idx, W0_1, W1_1, b01_1, b10_1,
           W0_2, W1_2, b01_2, b10_2, W_lin, b_lin):
    nd_p, ed_p = _deg(node_idx, edge_idx)
    ncard, ecard = _cards_call(nd_p, ed_p)
    s0_p, s1_p = _norm_sc(node_idx, edge_idx,
                          ncard.reshape(N_NODES), ecard.reshape(N_EDGES))
    d0_inv, d1_inv = _dinv_call(s0_p, s1_p)
    ncard = ncard.reshape(N_NODES, 1)
    ecard = ecard.reshape(N_EDGES, 1)
    d0_inv = d0_inv.reshape(N_NODES, 1)
    d1_inv = d1_inv.reshape(N_EDGES, 1)

    b01_1r = b01_1.reshape(1, HIDDEN)
    b10_1r = b10_1.reshape(1, HIDDEN)
    b01_2r = b01_2.reshape(1, HIDDEN)
    b10_2r = b10_2.reshape(1, HIDDEN)

    t1 = _prep_call(x_0, W0_1, ncard).reshape(2 * N_NODES, HALF)
    agg_e = _pass_n2e(t1, node_idx, edge_idx)
    t2 = _mid_call(agg_e, d1_inv, b01_1r, W1_1, ecard, 1000)
    t2 = t2.reshape(2 * N_EDGES, HALF)
    agg_v = _pass_e2n(t2, edge_idx, node_idx)

    t3 = _mid_call(agg_v, d0_inv, b10_1r, W0_2, ncard, 2000)
    t3 = t3.reshape(2 * N_NODES, HALF)
    agg_e2 = _pass_n2e(t3, node_idx, edge_idx)
    t4 = _mid_call(agg_e2, d1_inv, b01_2r, W1_2, ecard, 1000)
    t4 = t4.reshape(2 * N_EDGES, HALF)
    agg_v2 = _pass_e2n(t4, edge_idx, node_idx)

    out = _final_call(agg_v2, d0_inv, b10_2r, W_lin, b_lin.reshape(1, 1))
    return out.reshape(1)
